# baseline (speedup 1.0000x reference)
import functools

import jax
import jax.numpy as jnp
from jax import lax
from jax.experimental import pallas as pl
from jax.experimental.pallas import tpu as pltpu
from jax.experimental.pallas import tpu_sc as plsc

_EPS = 1e-6
_D = 96
_N = 1024
_P = 16384
_B = 16
_NW = 32
_CHUNK = (_B * _P) // _NW
_ROWS = _CHUNK // 128
_INV = 1.0 / (_P * _B)


_CT = _N // 128


def _tc_body(xg_ref, m_ref):
    eps = _EPS * _INV
    xi = xg_ref[0][:, :_D] * _INV
    xsq = xi * xi
    q_row = jnp.sum(xsq, axis=1, keepdims=True)
    s_row = jnp.sum(xi, axis=1, keepdims=True)
    qr = q_row + (2.0 * eps) * s_row + (_D * eps * eps)
    qc = q_row - (2.0 * eps) * s_row
    one_col = jnp.ones((_N, 1), jnp.float32)
    u = jnp.concatenate([-2.0 * xi, qr, one_col], axis=1)
    v = jnp.concatenate([xi, one_col, qc], axis=1)
    d2 = lax.dot_general(u, v, (((1,), (1,)), ((), ())),
                         preferred_element_type=jnp.float32)
    d2c = jnp.maximum(d2, 1e-36)
    pd = d2c * lax.rsqrt(d2c)
    ga = xg_ref[0][:, _D:_D + 1]
    a0 = (ga == 0.0).astype(jnp.float32)
    a1 = (ga == 1.0).astype(jnp.float32)
    a2 = (ga == 2.0).astype(jnp.float32)
    r_mat = jnp.concatenate([a0, a1, a2], axis=1)
    c_mat = jnp.concatenate([
        (2.0 * _INV) * a1 + (15.0 * _INV) * a2,
        (2.0 * _INV) * a0 + (17.0 * _INV) * a2,
        (15.0 * _INV) * a0 + (17.0 * _INV) * a1,
    ], axis=1)
    gtd = lax.dot_general(r_mat, c_mat, (((1,), (1,)), ((), ())),
                          preferred_element_type=jnp.float32)
    m_val = jnp.abs(gtd - pd)
    for tjj in range(_CT):
        m_ref[0, tjj] = m_val[:, tjj * 128:(tjj + 1) * 128]


_tc_call = pl.pallas_call(
    _tc_body,
    grid=(_B,),
    in_specs=[
        pl.BlockSpec((1, _N, _D + 1), lambda i: (i, 0, 0)),
    ],
    out_specs=pl.BlockSpec((1, _CT, _N, 128), lambda i: (i, 0, 0, 0)),
    out_shape=jax.ShapeDtypeStruct((_B, _CT, _N, 128), jnp.float32),
)


@functools.cache
def _sc_gather_fn():
    mesh = plsc.VectorSubcoreMesh(core_axis_name="c", subcore_axis_name="s")

    @functools.partial(
        pl.kernel,
        mesh=mesh,
        out_type=jax.ShapeDtypeStruct((_NW, 16), jnp.float32),
        scratch_types=[
            pltpu.VMEM((_CHUNK,), jnp.int32),
            pltpu.VMEM((_CHUNK,), jnp.int32),
            pltpu.VMEM((_CHUNK,), jnp.float32),
            pltpu.VMEM((_ROWS, 128), jnp.int32),
            pltpu.VMEM((_ROWS, 128), jnp.float32),
            pltpu.VMEM((16,), jnp.float32),
            pltpu.SemaphoreType.DMA,
        ],
    )
    def _sc_gather(pp_hbm, w_hbm, m_hbm, out_hbm,
                   a_v, c_v, w_v, idx_v, g_v, acc_v, sem):
        wid = lax.axis_index("s") * 2 + lax.axis_index("c")
        samp = wid // 2
        off = (wid % 2) * _CHUNK
        pltpu.sync_copy(pp_hbm.at[samp, 0, pl.ds(off, _CHUNK)], a_v)
        pltpu.sync_copy(pp_hbm.at[samp, 1, pl.ds(off, _CHUNK)], c_v)
        pltpu.sync_copy(w_hbm.at[samp, pl.ds(off, _CHUNK)], w_v)
        samp_base = samp * (_N * _N)

        def fire(r, carry):
            for k in range(8):
                sl = pl.ds(k * 16, 16)
                av = a_v[pl.ds(r * 128 + k * 16, 16)]
                cv = c_v[pl.ds(r * 128 + k * 16, 16)]
                idx_v[r, sl] = (samp_base
                                + ((cv >> 7) << 17) + (av << 7)
                                + (cv & 127))
            pltpu.async_copy(m_hbm.at[idx_v.at[r]], g_v.at[r], sem)
            return carry

        lax.fori_loop(0, _ROWS, fire, 0)

        def drain(r, acc):
            pltpu.make_async_copy(m_hbm.at[idx_v.at[r]], g_v.at[r], sem).wait()
            for k in range(8):
                acc = (acc + g_v[r, pl.ds(k * 16, 16)]
                       * w_v[pl.ds(r * 128 + k * 16, 16)])
            return acc

        acc = lax.fori_loop(0, _ROWS, drain, jnp.zeros((16,), jnp.float32))
        acc_v[...] = acc
        pltpu.sync_copy(acc_v, out_hbm.at[wid])

    return _sc_gather


def kernel(x, patch_pair, patch_gt, patch_pair_weight):
    pp = patch_pair.astype(jnp.int32)
    xg = jnp.concatenate(
        [x, patch_gt.astype(jnp.float32)[:, :, None]], axis=2)
    m = _tc_call(xg)
    partials = _sc_gather_fn()(pp, patch_pair_weight,
                               m.reshape(_B * _N * _N))
    return jnp.sum(partials)


# speedup vs baseline: 1.2010x; 1.2010x over previous
import functools

import jax
import jax.numpy as jnp
from jax import lax
from jax.experimental import pallas as pl
from jax.experimental.pallas import tpu as pltpu
from jax.experimental.pallas import tpu_sc as plsc

_EPS = 1e-6
_D = 96
_N = 1024
_P = 16384
_B = 16
_NW = 32
_CHUNK = (_B * _P) // _NW
_ROWS = _CHUNK // 128
_INV = 1.0 / (_P * _B)


_CT = _N // 128


def _tc_body(x_ref, gt_ref, m_ref):
    eps = _EPS * _INV
    xi = x_ref[0] * _INV
    xsq = xi * xi
    q_row = jnp.sum(xsq, axis=1, keepdims=True)
    s_row = jnp.sum(xi, axis=1, keepdims=True)
    qr = q_row + (2.0 * eps) * s_row + (_D * eps * eps)
    qc = q_row - (2.0 * eps) * s_row
    one_col = jnp.ones((_N, 1), jnp.float32)
    u = jnp.concatenate([-2.0 * xi, qr, one_col], axis=1)
    v = jnp.concatenate([xi, one_col, qc], axis=1)
    d2 = lax.dot_general(u, v, (((1,), (1,)), ((), ())),
                         preferred_element_type=jnp.float32)
    d2c = jnp.maximum(d2, 1e-36)
    pd = d2c * lax.rsqrt(d2c)
    ga = gt_ref[0]
    a0 = (ga == 0.0).astype(jnp.float32)
    a1 = (ga == 1.0).astype(jnp.float32)
    a2 = (ga == 2.0).astype(jnp.float32)
    p_mat = jnp.concatenate([a0, a1, a2], axis=0)
    q_mat = jnp.concatenate([
        (2.0 * _INV) * a1 + (15.0 * _INV) * a2,
        (2.0 * _INV) * a0 + (17.0 * _INV) * a2,
        (15.0 * _INV) * a0 + (17.0 * _INV) * a1,
    ], axis=0)
    gtd = lax.dot_general(p_mat, q_mat, (((0,), (0,)), ((), ())),
                          preferred_element_type=jnp.float32)
    m_val = jnp.abs(gtd - pd)
    for tjj in range(_CT):
        m_ref[0, tjj] = m_val[:, tjj * 128:(tjj + 1) * 128]


_tc_call = pl.pallas_call(
    _tc_body,
    grid=(_B,),
    in_specs=[
        pl.BlockSpec((1, _N, _D), lambda i: (i, 0, 0)),
        pl.BlockSpec((1, 1, _N), lambda i: (i, 0, 0)),
    ],
    out_specs=pl.BlockSpec((1, _CT, _N, 128), lambda i: (i, 0, 0, 0)),
    out_shape=jax.ShapeDtypeStruct((_B, _CT, _N, 128), jnp.float32),
)


@functools.cache
def _sc_gather_fn():
    mesh = plsc.VectorSubcoreMesh(core_axis_name="c", subcore_axis_name="s")

    @functools.partial(
        pl.kernel,
        mesh=mesh,
        out_type=jax.ShapeDtypeStruct((_NW, 16), jnp.float32),
        scratch_types=[
            pltpu.VMEM((_CHUNK,), jnp.int32),
            pltpu.VMEM((_CHUNK,), jnp.int32),
            pltpu.VMEM((_CHUNK,), jnp.float32),
            pltpu.VMEM((_ROWS, 128), jnp.int32),
            pltpu.VMEM((_ROWS, 128), jnp.float32),
            pltpu.VMEM((16,), jnp.float32),
            pltpu.SemaphoreType.DMA,
        ],
    )
    def _sc_gather(pp_hbm, w_hbm, m_hbm, out_hbm,
                   a_v, c_v, w_v, idx_v, g_v, acc_v, sem):
        wid = lax.axis_index("s") * 2 + lax.axis_index("c")
        samp = wid // 2
        off = (wid % 2) * _CHUNK
        pltpu.sync_copy(pp_hbm.at[samp, 0, pl.ds(off, _CHUNK)], a_v)
        pltpu.sync_copy(pp_hbm.at[samp, 1, pl.ds(off, _CHUNK)], c_v)
        pltpu.sync_copy(w_hbm.at[samp, pl.ds(off, _CHUNK)], w_v)
        samp_base = samp * (_N * _N)

        def fire(r, carry):
            for k in range(8):
                sl = pl.ds(k * 16, 16)
                av = a_v[pl.ds(r * 128 + k * 16, 16)]
                cv = c_v[pl.ds(r * 128 + k * 16, 16)]
                idx_v[r, sl] = (samp_base
                                + ((cv >> 7) << 17) + (av << 7)
                                + (cv & 127))
            pltpu.async_copy(m_hbm.at[idx_v.at[r]], g_v.at[r], sem)
            return carry

        lax.fori_loop(0, _ROWS, fire, 0)

        def drain(r, acc):
            pltpu.make_async_copy(m_hbm.at[idx_v.at[r]], g_v.at[r], sem).wait()
            for k in range(8):
                acc = (acc + g_v[r, pl.ds(k * 16, 16)]
                       * w_v[pl.ds(r * 128 + k * 16, 16)])
            return acc

        acc = lax.fori_loop(0, _ROWS, drain, jnp.zeros((16,), jnp.float32))
        acc_v[...] = acc
        pltpu.sync_copy(acc_v, out_hbm.at[wid])

    return _sc_gather


def kernel(x, patch_pair, patch_gt, patch_pair_weight):
    pp = patch_pair.astype(jnp.int32)
    gt1r = patch_gt.astype(jnp.float32).reshape(_B, 1, _N)
    m = _tc_call(x, gt1r)
    partials = _sc_gather_fn()(pp, patch_pair_weight,
                               m.reshape(_B * _N * _N))
    return jnp.sum(partials)
